# baseline (speedup 1.0000x reference)
import functools

import jax
import jax.numpy as jnp
from jax import lax
from jax.experimental import pallas as pl
from jax.experimental.pallas import tpu as pltpu
from jax.experimental.pallas import tpu_sc as plsc

B, G, N = 64, 16, 32768
V = N // 16
NW = 32
ROWS_PER_W = B // NW
H = N // 2
V2 = V // 2
U2 = 4
KA = (1 << 17) - 1


def _keys_kernel(x_ref, k_ref):
    xb = x_ref[0]
    w = jnp.mean(xb, axis=0, keepdims=True)
    b = lax.bitcast_convert_type(w, jnp.int32)
    kasc = jnp.where(b < 0, jnp.bitwise_not(b),
                     jnp.bitwise_xor(b, jnp.int32(-(2 ** 31))))
    k_ref[0] = jnp.bitwise_not(kasc)


def _make_keys(x):
    out = pl.pallas_call(
        _keys_kernel,
        grid=(B,),
        in_specs=[pl.BlockSpec((1, G, N), lambda i: (i, 0, 0))],
        out_specs=pl.BlockSpec((1, 1, N), lambda i: (i, 0, 0)),
        out_shape=jax.ShapeDtypeStruct((B, 1, N), jnp.int32),
    )(x)
    return out.reshape(B, N)


def _sc_argsort(keys_s):
    mesh = plsc.VectorSubcoreMesh(core_axis_name="c", subcore_axis_name="s")

    @functools.partial(
        pl.kernel,
        mesh=mesh,
        out_type=jax.ShapeDtypeStruct((B, N), jnp.int32),
        compiler_params=pltpu.CompilerParams(needs_layout_passes=False),
        scratch_types=[
            pltpu.VMEM((N,), jnp.int32),
            pltpu.VMEM((N,), jnp.int32),
            pltpu.VMEM((N,), jnp.int32),
            pltpu.VMEM((4096,), jnp.int32),
            pltpu.VMEM((4096,), jnp.int32),
            pltpu.VMEM((4096,), jnp.int32),
            pltpu.VMEM((4096,), jnp.int32),
        ],
    )
    def body(keys_hbm, out_hbm, keys_v, pay_a, pay_b, ha0, ha1, hb0, hb1):
        cid = lax.axis_index("c")
        sid = lax.axis_index("s")
        wid = sid * 2 + cid
        lane = lax.broadcasted_iota(jnp.int32, (16,), 0)
        ones = jnp.full((16,), 1, jnp.int32)
        zeros = jnp.zeros((16,), jnp.int32)

        def dig(k, sh):
            if sh:
                k = k >> sh
            return jnp.bitwise_and(k, jnp.int32(255))

        def slot_of(d):
            return jnp.bitwise_or(d << 4, lane)

        def fetch_add(h, slot):
            q = plsc.load_gather(h, [slot])
            plsc.store_scatter(h, [slot], q + ones)
            return q

        def remap(q):
            return jnp.bitwise_or((jnp.bitwise_and(q, jnp.int32(V - 1))) << 4,
                                  q >> 11)

        def zero_pair(h0, h1):
            def zbody(i, c):
                for u in range(2):
                    h0[pl.ds(i * 32 + u * 16, 16)] = zeros
                    h1[pl.ds(i * 32 + u * 16, 16)] = zeros
                return c
            return zbody

        def make_scan(s0, s1, z0, z1):
            def scan_body(i, carry):
                h0 = s0[pl.ds(i * 16, 16)]
                h1 = s1[pl.ds(i * 16, 16)]
                hs = h0 + h1
                inc = plsc.cumsum(hs)
                excl = inc - hs + carry
                s0[pl.ds(i * 16, 16)] = excl
                s1[pl.ds(i * 16, 16)] = excl + h0
                z0[pl.ds(i * 16, 16)] = zeros
                z1[pl.ds(i * 16, 16)] = zeros
                return carry + inc[15]
            return scan_body

        def make_hsweep(pay_src, h0, h1):
            def hs(i, c):
                for u in range(U2):
                    pay = pay_src[pl.ds(i * 16 * U2 + u * 16, 16)]
                    plsc.addupdate_scatter(h0, [slot_of(pay >> 17)], ones)
                    pay = pay_src[pl.ds(H + i * 16 * U2 + u * 16, 16)]
                    plsc.addupdate_scatter(h1, [slot_of(pay >> 17)], ones)
                return c
            return hs

        lax.fori_loop(0, 128, zero_pair(ha0, ha1), 0)

        for rr in range(ROWS_PER_W):
            r = wid * ROWS_PER_W + rr
            pltpu.sync_copy(keys_hbm.at[r], keys_v)

            def sw0(i, c):
                for u in range(U2):
                    k = keys_v[pl.ds(i * 16 * U2 + u * 16, 16)]
                    plsc.addupdate_scatter(ha0, [slot_of(dig(k, 0))], ones)
                    k = keys_v[pl.ds(H + i * 16 * U2 + u * 16, 16)]
                    plsc.addupdate_scatter(ha1, [slot_of(dig(k, 0))], ones)
                return c
            lax.fori_loop(0, V2 // U2, sw0, 0)

            lax.fori_loop(0, 256, make_scan(ha0, ha1, hb0, hb1), jnp.int32(0))

            def p0(i, c):
                for u in range(U2):
                    for (base, h) in ((0, ha0), (H, ha1)):
                        a0 = base + i * 16 * U2 + u * 16
                        k = keys_v[pl.ds(a0, 16)]
                        q = fetch_add(h, slot_of(dig(k, 0)))
                        pay = jnp.bitwise_or(dig(k, 8) << 17, lane + a0)
                        plsc.store_scatter(pay_a, [remap(q)], pay)
                return c
            lax.fori_loop(0, V2 // U2, p0, 0)

            lax.fori_loop(0, V2 // U2, make_hsweep(pay_a, hb0, hb1), 0)
            lax.fori_loop(0, 256, make_scan(hb0, hb1, ha0, ha1), jnp.int32(0))

            def p1(i, c):
                for u in range(U2):
                    for (base, h) in ((0, hb0), (H, hb1)):
                        a0 = base + i * 16 * U2 + u * 16
                        pay = pay_a[pl.ds(a0, 16)]
                        q = fetch_add(h, slot_of(pay >> 17))
                        ka = jnp.bitwise_and(pay, jnp.int32(KA))
                        k = plsc.load_gather(keys_v, [ka])
                        pay2 = jnp.bitwise_or(dig(k, 16) << 17, ka)
                        plsc.store_scatter(pay_b, [remap(q)], pay2)
                return c
            lax.fori_loop(0, V2 // U2, p1, 0)

            lax.fori_loop(0, V2 // U2, make_hsweep(pay_b, ha0, ha1), 0)
            lax.fori_loop(0, 256, make_scan(ha0, ha1, hb0, hb1), jnp.int32(0))

            def p2(i, c):
                for u in range(U2):
                    for (base, h) in ((0, ha0), (H, ha1)):
                        a0 = base + i * 16 * U2 + u * 16
                        pay = pay_b[pl.ds(a0, 16)]
                        q = fetch_add(h, slot_of(pay >> 17))
                        ka = jnp.bitwise_and(pay, jnp.int32(KA))
                        k = plsc.load_gather(keys_v, [ka])
                        pay2 = jnp.bitwise_or(dig(k, 24) << 17, ka)
                        plsc.store_scatter(pay_a, [remap(q)], pay2)
                return c
            lax.fori_loop(0, V2 // U2, p2, 0)

            lax.fori_loop(0, V2 // U2, make_hsweep(pay_a, hb0, hb1), 0)
            lax.fori_loop(0, 256, make_scan(hb0, hb1, ha0, ha1), jnp.int32(0))

            def p3(i, c):
                for u in range(U2):
                    for (base, h) in ((0, hb0), (H, hb1)):
                        a0 = base + i * 16 * U2 + u * 16
                        pay = pay_a[pl.ds(a0, 16)]
                        q = fetch_add(h, slot_of(pay >> 17))
                        ka = jnp.bitwise_and(pay, jnp.int32(KA))
                        orig = jnp.bitwise_or(
                            (jnp.bitwise_and(ka, jnp.int32(15))) << 11,
                            ka >> 4)
                        plsc.store_scatter(pay_b, [q], orig)
                return c
            lax.fori_loop(0, V2 // U2, p3, 0)

            if rr + 1 < ROWS_PER_W:
                lax.fori_loop(0, 128, zero_pair(hb0, hb1), 0)

            pltpu.sync_copy(pay_b, out_hbm.at[r])

    return body(keys_s)


def kernel(x):
    keys = _make_keys(x)
    keys_s = keys.reshape(B, 16, V).swapaxes(1, 2).reshape(B, N)
    return _sc_argsort(keys_s)


# speedup vs baseline: 1.0212x; 1.0212x over previous
import functools

import jax
import jax.numpy as jnp
from jax import lax
from jax.experimental import pallas as pl
from jax.experimental.pallas import tpu as pltpu
from jax.experimental.pallas import tpu_sc as plsc

B, G, N = 64, 16, 32768
V = N // 16
NW = 32
ROWS_PER_W = B // NW
U = 8
KA = (1 << 15) - 1


def _keys_kernel(x_ref, k_ref):
    xb = x_ref[0]
    w = jnp.mean(xb, axis=0, keepdims=True)
    b = lax.bitcast_convert_type(w, jnp.int32)
    kasc = jnp.where(b < 0, jnp.bitwise_not(b),
                     jnp.bitwise_xor(b, jnp.int32(-(2 ** 31))))
    k_ref[0] = jnp.bitwise_not(kasc)


def _make_keys(x):
    out = pl.pallas_call(
        _keys_kernel,
        grid=(B,),
        in_specs=[pl.BlockSpec((1, G, N), lambda i: (i, 0, 0))],
        out_specs=pl.BlockSpec((1, 1, N), lambda i: (i, 0, 0)),
        out_shape=jax.ShapeDtypeStruct((B, 1, N), jnp.int32),
    )(x)
    return out.reshape(B, N)


def _sc_argsort(keys_s):
    mesh = plsc.VectorSubcoreMesh(core_axis_name="c", subcore_axis_name="s")

    @functools.partial(
        pl.kernel,
        mesh=mesh,
        out_type=jax.ShapeDtypeStruct((B, N), jnp.int32),
        compiler_params=pltpu.CompilerParams(needs_layout_passes=False),
        scratch_types=[
            pltpu.VMEM((N,), jnp.int32),
            pltpu.VMEM((N,), jnp.int32),
            pltpu.VMEM((N,), jnp.int32),
            pltpu.VMEM((4096,), jnp.int32),
            pltpu.VMEM((4096,), jnp.int32),
        ],
    )
    def body(keys_hbm, out_hbm, keys_v, pay_a, pay_b, hist_a, hist_b):
        cid = lax.axis_index("c")
        sid = lax.axis_index("s")
        wid = sid * 2 + cid
        lane = lax.broadcasted_iota(jnp.int32, (16,), 0)
        ones = jnp.full((16,), 1, jnp.int32)
        zeros = jnp.zeros((16,), jnp.int32)

        def dig(k, sh):
            if sh:
                k = k >> sh
            return jnp.bitwise_and(k, jnp.int32(255))

        def slot_of(d):
            return jnp.bitwise_or(d << 4, lane)

        def fetch_add(h, slot):
            q = plsc.load_gather(h, [slot])
            plsc.store_scatter(h, [slot], q + ones)
            return q

        def remap(q):
            return jnp.bitwise_or((jnp.bitwise_and(q, jnp.int32(V - 1))) << 4,
                                  q >> 11)

        def pdig(pay):
            return jnp.bitwise_and(pay >> 15, jnp.int32(255))

        def zero_hist(h):
            def zbody(i, c):
                for u in range(U):
                    h[pl.ds(i * 16 * U + u * 16, 16)] = zeros
                return c
            return zbody

        def make_scan(h_scan, h_zero):
            def scan_body(i, carry):
                hh = h_scan[pl.ds(i * 16, 16)]
                inc = plsc.cumsum(hh)
                h_scan[pl.ds(i * 16, 16)] = inc - hh + carry
                h_zero[pl.ds(i * 16, 16)] = zeros
                return carry + inc[15]
            return scan_body

        def make_hsweep(pay_src, h):
            def hs(i, c):
                for u in range(U):
                    pay = pay_src[pl.ds(i * 16 * U + u * 16, 16)]
                    plsc.addupdate_scatter(h, [slot_of(pdig(pay))], ones)
                return c
            return hs

        lax.fori_loop(0, 256 // U, zero_hist(hist_a), 0)

        for rr in range(ROWS_PER_W):
            r = wid * ROWS_PER_W + rr
            pltpu.sync_copy(keys_hbm.at[r], keys_v)

            def sw0(i, c):
                for u in range(U):
                    k = keys_v[pl.ds(i * 16 * U + u * 16, 16)]
                    plsc.addupdate_scatter(hist_a, [slot_of(dig(k, 0))], ones)
                return c
            lax.fori_loop(0, V // U, sw0, 0)

            lax.fori_loop(0, 256, make_scan(hist_a, hist_b), jnp.int32(0))

            def p0(i, c):
                for u in range(U):
                    a0 = i * 16 * U + u * 16
                    k = keys_v[pl.ds(a0, 16)]
                    q = fetch_add(hist_a, slot_of(dig(k, 0)))
                    pay = jnp.bitwise_or(
                        jnp.bitwise_or(dig(k, 16) << 23, dig(k, 8) << 15),
                        lane + a0)
                    plsc.store_scatter(pay_a, [remap(q)], pay)
                return c
            lax.fori_loop(0, V // U, p0, 0)

            lax.fori_loop(0, V // U, make_hsweep(pay_a, hist_b), 0)
            lax.fori_loop(0, 256, make_scan(hist_b, hist_a), jnp.int32(0))

            def p1(i, c):
                for u in range(U):
                    a0 = i * 16 * U + u * 16
                    pay = pay_a[pl.ds(a0, 16)]
                    q = fetch_add(hist_b, slot_of(pdig(pay)))
                    ka = jnp.bitwise_and(pay, jnp.int32(KA))
                    k = plsc.load_gather(keys_v, [ka])
                    pay2 = jnp.bitwise_or(
                        jnp.bitwise_or(dig(k, 24) << 23,
                                       (pay >> 23) << 15), ka)
                    plsc.store_scatter(pay_b, [remap(q)], pay2)
                return c
            lax.fori_loop(0, V // U, p1, 0)

            lax.fori_loop(0, V // U, make_hsweep(pay_b, hist_a), 0)
            lax.fori_loop(0, 256, make_scan(hist_a, hist_b), jnp.int32(0))

            def p2(i, c):
                for u in range(U):
                    a0 = i * 16 * U + u * 16
                    pay = pay_b[pl.ds(a0, 16)]
                    q = fetch_add(hist_a, slot_of(pdig(pay)))
                    ka = jnp.bitwise_and(pay, jnp.int32(KA))
                    pay2 = jnp.bitwise_or((pay >> 23) << 15, ka)
                    plsc.store_scatter(pay_a, [remap(q)], pay2)
                return c
            lax.fori_loop(0, V // U, p2, 0)

            lax.fori_loop(0, V // U, make_hsweep(pay_a, hist_b), 0)
            lax.fori_loop(0, 256, make_scan(hist_b, hist_a), jnp.int32(0))

            def p3(i, c):
                for u in range(U):
                    a0 = i * 16 * U + u * 16
                    pay = pay_a[pl.ds(a0, 16)]
                    q = fetch_add(hist_b, slot_of(pdig(pay)))
                    ka = jnp.bitwise_and(pay, jnp.int32(KA))
                    orig = jnp.bitwise_or(
                        (jnp.bitwise_and(ka, jnp.int32(15))) << 11, ka >> 4)
                    plsc.store_scatter(pay_b, [q], orig)
                return c
            lax.fori_loop(0, V // U, p3, 0)

            if rr + 1 < ROWS_PER_W:
                lax.fori_loop(0, 256 // U, zero_hist(hist_b), 0)

            pltpu.sync_copy(pay_b, out_hbm.at[r])

    return body(keys_s)


def kernel(x):
    keys = _make_keys(x)
    keys_s = keys.reshape(B, 16, V).swapaxes(1, 2).reshape(B, N)
    return _sc_argsort(keys_s)


# speedup vs baseline: 1.0279x; 1.0066x over previous
import functools

import jax
import jax.numpy as jnp
from jax import lax
from jax.experimental import pallas as pl
from jax.experimental.pallas import tpu as pltpu
from jax.experimental.pallas import tpu_sc as plsc

B, G, N = 64, 16, 32768
V = N // 16
NW = 32
ROWS_PER_W = B // NW
U = 8
KA = (1 << 15) - 1


def _keys_kernel(x_ref, k_ref):
    xb = x_ref[0]
    w = jnp.mean(xb, axis=0, keepdims=True)
    b = lax.bitcast_convert_type(w, jnp.int32)
    kasc = jnp.where(b < 0, jnp.bitwise_not(b),
                     jnp.bitwise_xor(b, jnp.int32(-(2 ** 31))))
    k_ref[0] = jnp.bitwise_not(kasc)


def _make_keys(x):
    out = pl.pallas_call(
        _keys_kernel,
        grid=(B,),
        in_specs=[pl.BlockSpec((1, G, N), lambda i: (i, 0, 0))],
        out_specs=pl.BlockSpec((1, 1, N), lambda i: (i, 0, 0)),
        out_shape=jax.ShapeDtypeStruct((B, 1, N), jnp.int32),
    )(x)
    return out.reshape(B, N)


def _sc_argsort(keys_s):
    mesh = plsc.VectorSubcoreMesh(core_axis_name="c", subcore_axis_name="s")

    @functools.partial(
        pl.kernel,
        mesh=mesh,
        out_type=jax.ShapeDtypeStruct((B, N), jnp.int32),
        compiler_params=pltpu.CompilerParams(needs_layout_passes=False),
        scratch_types=[
            pltpu.VMEM((N,), jnp.int32),
            pltpu.VMEM((N,), jnp.int32),
            pltpu.VMEM((N,), jnp.int32),
            pltpu.VMEM((4096,), jnp.int32),
            pltpu.VMEM((4096,), jnp.int32),
            pltpu.SemaphoreType.DMA,
            pltpu.SemaphoreType.DMA,
        ],
    )
    def body(keys_hbm, out_hbm, keys_v, pay_a, pay_b, hist_a, hist_b,
             ksem, osem):
        cid = lax.axis_index("c")
        sid = lax.axis_index("s")
        wid = sid * 2 + cid
        lane = lax.broadcasted_iota(jnp.int32, (16,), 0)
        ones = jnp.full((16,), 1, jnp.int32)
        zeros = jnp.zeros((16,), jnp.int32)

        def dig(k, sh):
            if sh:
                k = k >> sh
            return jnp.bitwise_and(k, jnp.int32(255))

        def slot_of(d):
            return jnp.bitwise_or(d << 4, lane)

        def fetch_add(h, slot):
            q = plsc.load_gather(h, [slot])
            plsc.store_scatter(h, [slot], q + ones)
            return q

        def remap(q):
            return jnp.bitwise_or((jnp.bitwise_and(q, jnp.int32(V - 1))) << 4,
                                  q >> 11)

        def pdig(pay):
            return jnp.bitwise_and(pay >> 15, jnp.int32(255))

        def zero_hist(h):
            def zbody(i, c):
                for u in range(U):
                    h[pl.ds(i * 16 * U + u * 16, 16)] = zeros
                return c
            return zbody

        def make_scan(h_scan, h_zero):
            def scan_body(i, carry):
                hh = h_scan[pl.ds(i * 16, 16)]
                inc = plsc.cumsum(hh)
                h_scan[pl.ds(i * 16, 16)] = inc - hh + carry
                h_zero[pl.ds(i * 16, 16)] = zeros
                return carry + inc[15]
            return scan_body

        def make_hsweep(pay_src, h):
            def hs(i, c):
                for u in range(U):
                    pay = pay_src[pl.ds(i * 16 * U + u * 16, 16)]
                    plsc.addupdate_scatter(h, [slot_of(pdig(pay))], ones)
                return c
            return hs

        kcopy = pltpu.async_copy(keys_hbm.at[wid * ROWS_PER_W], keys_v, ksem)
        ocopy = None
        lax.fori_loop(0, 256 // U, zero_hist(hist_a), 0)

        for rr in range(ROWS_PER_W):
            r = wid * ROWS_PER_W + rr
            kcopy.wait()

            def sw0(i, c):
                for u in range(U):
                    k = keys_v[pl.ds(i * 16 * U + u * 16, 16)]
                    plsc.addupdate_scatter(hist_a, [slot_of(dig(k, 0))], ones)
                return c
            lax.fori_loop(0, V // U, sw0, 0)

            lax.fori_loop(0, 256, make_scan(hist_a, hist_b), jnp.int32(0))

            def p0(i, c):
                for u in range(U):
                    a0 = i * 16 * U + u * 16
                    k = keys_v[pl.ds(a0, 16)]
                    q = fetch_add(hist_a, slot_of(dig(k, 0)))
                    pay = jnp.bitwise_or(
                        jnp.bitwise_or(dig(k, 16) << 23, dig(k, 8) << 15),
                        lane + a0)
                    plsc.store_scatter(pay_a, [remap(q)], pay)
                return c
            lax.fori_loop(0, V // U, p0, 0)

            lax.fori_loop(0, V // U, make_hsweep(pay_a, hist_b), 0)
            lax.fori_loop(0, 256, make_scan(hist_b, hist_a), jnp.int32(0))

            if ocopy is not None:
                ocopy.wait()

            def p1(i, c):
                for u in range(U):
                    a0 = i * 16 * U + u * 16
                    pay = pay_a[pl.ds(a0, 16)]
                    q = fetch_add(hist_b, slot_of(pdig(pay)))
                    ka = jnp.bitwise_and(pay, jnp.int32(KA))
                    k = plsc.load_gather(keys_v, [ka])
                    pay2 = jnp.bitwise_or(
                        jnp.bitwise_or(dig(k, 24) << 23,
                                       (pay >> 23) << 15), ka)
                    plsc.store_scatter(pay_b, [remap(q)], pay2)
                return c
            lax.fori_loop(0, V // U, p1, 0)

            if rr + 1 < ROWS_PER_W:
                kcopy = pltpu.async_copy(keys_hbm.at[r + 1], keys_v, ksem)

            lax.fori_loop(0, V // U, make_hsweep(pay_b, hist_a), 0)
            lax.fori_loop(0, 256, make_scan(hist_a, hist_b), jnp.int32(0))

            def p2(i, c):
                for u in range(U):
                    a0 = i * 16 * U + u * 16
                    pay = pay_b[pl.ds(a0, 16)]
                    q = fetch_add(hist_a, slot_of(pdig(pay)))
                    ka = jnp.bitwise_and(pay, jnp.int32(KA))
                    pay2 = jnp.bitwise_or((pay >> 23) << 15, ka)
                    plsc.store_scatter(pay_a, [remap(q)], pay2)
                return c
            lax.fori_loop(0, V // U, p2, 0)

            lax.fori_loop(0, V // U, make_hsweep(pay_a, hist_b), 0)
            lax.fori_loop(0, 256, make_scan(hist_b, hist_a), jnp.int32(0))

            def p3(i, c):
                for u in range(U):
                    a0 = i * 16 * U + u * 16
                    pay = pay_a[pl.ds(a0, 16)]
                    q = fetch_add(hist_b, slot_of(pdig(pay)))
                    ka = jnp.bitwise_and(pay, jnp.int32(KA))
                    orig = jnp.bitwise_or(
                        (jnp.bitwise_and(ka, jnp.int32(15))) << 11, ka >> 4)
                    plsc.store_scatter(pay_b, [q], orig)
                return c
            lax.fori_loop(0, V // U, p3, 0)

            if rr + 1 < ROWS_PER_W:
                lax.fori_loop(0, 256 // U, zero_hist(hist_b), 0)

            ocopy = pltpu.async_copy(pay_b, out_hbm.at[r], osem)

        ocopy.wait()

    return body(keys_s)


def kernel(x):
    keys = _make_keys(x)
    keys_s = keys.reshape(B, 16, V).swapaxes(1, 2).reshape(B, N)
    return _sc_argsort(keys_s)


# speedup vs baseline: 1.0293x; 1.0013x over previous
import functools

import jax
import jax.numpy as jnp
from jax import lax
from jax.experimental import pallas as pl
from jax.experimental.pallas import tpu as pltpu
from jax.experimental.pallas import tpu_sc as plsc

B, G, N = 64, 16, 32768
V = N // 16
NW = 32
ROWS_PER_W = B // NW
U = 16
KA = (1 << 15) - 1


def _keys_kernel(x_ref, k_ref):
    xb = x_ref[0]
    w = jnp.mean(xb, axis=0, keepdims=True)
    b = lax.bitcast_convert_type(w, jnp.int32)
    kasc = jnp.where(b < 0, jnp.bitwise_not(b),
                     jnp.bitwise_xor(b, jnp.int32(-(2 ** 31))))
    k_ref[0] = jnp.bitwise_not(kasc)


def _make_keys(x):
    out = pl.pallas_call(
        _keys_kernel,
        grid=(B,),
        in_specs=[pl.BlockSpec((1, G, N), lambda i: (i, 0, 0))],
        out_specs=pl.BlockSpec((1, 1, N), lambda i: (i, 0, 0)),
        out_shape=jax.ShapeDtypeStruct((B, 1, N), jnp.int32),
    )(x)
    return out.reshape(B, N)


def _sc_argsort(keys_s):
    mesh = plsc.VectorSubcoreMesh(core_axis_name="c", subcore_axis_name="s")

    @functools.partial(
        pl.kernel,
        mesh=mesh,
        out_type=jax.ShapeDtypeStruct((B, N), jnp.int32),
        compiler_params=pltpu.CompilerParams(needs_layout_passes=False),
        scratch_types=[
            pltpu.VMEM((N,), jnp.int32),
            pltpu.VMEM((N,), jnp.int32),
            pltpu.VMEM((N,), jnp.int32),
            pltpu.VMEM((4096,), jnp.int32),
            pltpu.VMEM((4096,), jnp.int32),
            pltpu.SemaphoreType.DMA,
            pltpu.SemaphoreType.DMA,
        ],
    )
    def body(keys_hbm, out_hbm, keys_v, pay_a, pay_b, hist_a, hist_b,
             ksem, osem):
        cid = lax.axis_index("c")
        sid = lax.axis_index("s")
        wid = sid * 2 + cid
        lane = lax.broadcasted_iota(jnp.int32, (16,), 0)
        ones = jnp.full((16,), 1, jnp.int32)
        zeros = jnp.zeros((16,), jnp.int32)

        def dig(k, sh):
            if sh:
                k = k >> sh
            return jnp.bitwise_and(k, jnp.int32(255))

        def slot_of(d):
            return jnp.bitwise_or(d << 4, lane)

        def fetch_add(h, slot):
            q = plsc.load_gather(h, [slot])
            plsc.store_scatter(h, [slot], q + ones)
            return q

        def remap(q):
            return jnp.bitwise_or((jnp.bitwise_and(q, jnp.int32(V - 1))) << 4,
                                  q >> 11)

        def pdig(pay):
            return jnp.bitwise_and(pay >> 15, jnp.int32(255))

        def zero_hist(h):
            def zbody(i, c):
                for u in range(U):
                    h[pl.ds(i * 16 * U + u * 16, 16)] = zeros
                return c
            return zbody

        def make_scan(h_scan, h_zero):
            def scan_body(i, carry):
                hh = h_scan[pl.ds(i * 16, 16)]
                inc = plsc.cumsum(hh)
                h_scan[pl.ds(i * 16, 16)] = inc - hh + carry
                h_zero[pl.ds(i * 16, 16)] = zeros
                return carry + inc[15]
            return scan_body

        def make_hsweep(pay_src, h):
            def hs(i, c):
                for u in range(U):
                    pay = pay_src[pl.ds(i * 16 * U + u * 16, 16)]
                    plsc.addupdate_scatter(h, [slot_of(pdig(pay))], ones)
                return c
            return hs

        kcopy = pltpu.async_copy(keys_hbm.at[wid * ROWS_PER_W], keys_v, ksem)
        ocopy = None
        lax.fori_loop(0, 256 // U, zero_hist(hist_a), 0)

        for rr in range(ROWS_PER_W):
            r = wid * ROWS_PER_W + rr
            kcopy.wait()

            def sw0(i, c):
                for u in range(U):
                    k = keys_v[pl.ds(i * 16 * U + u * 16, 16)]
                    plsc.addupdate_scatter(hist_a, [slot_of(dig(k, 0))], ones)
                return c
            lax.fori_loop(0, V // U, sw0, 0)

            lax.fori_loop(0, 256, make_scan(hist_a, hist_b), jnp.int32(0))

            def p0(i, c):
                for u in range(U):
                    a0 = i * 16 * U + u * 16
                    k = keys_v[pl.ds(a0, 16)]
                    q = fetch_add(hist_a, slot_of(dig(k, 0)))
                    pay = jnp.bitwise_or(
                        jnp.bitwise_or(dig(k, 16) << 23, dig(k, 8) << 15),
                        lane + a0)
                    plsc.store_scatter(pay_a, [remap(q)], pay)
                return c
            lax.fori_loop(0, V // U, p0, 0)

            lax.fori_loop(0, V // U, make_hsweep(pay_a, hist_b), 0)
            lax.fori_loop(0, 256, make_scan(hist_b, hist_a), jnp.int32(0))

            if ocopy is not None:
                ocopy.wait()

            def p1(i, c):
                for u in range(U):
                    a0 = i * 16 * U + u * 16
                    pay = pay_a[pl.ds(a0, 16)]
                    q = fetch_add(hist_b, slot_of(pdig(pay)))
                    ka = jnp.bitwise_and(pay, jnp.int32(KA))
                    k = plsc.load_gather(keys_v, [ka])
                    pay2 = jnp.bitwise_or(
                        jnp.bitwise_or(dig(k, 24) << 23,
                                       (pay >> 23) << 15), ka)
                    plsc.store_scatter(pay_b, [remap(q)], pay2)
                return c
            lax.fori_loop(0, V // U, p1, 0)

            if rr + 1 < ROWS_PER_W:
                kcopy = pltpu.async_copy(keys_hbm.at[r + 1], keys_v, ksem)

            lax.fori_loop(0, V // U, make_hsweep(pay_b, hist_a), 0)
            lax.fori_loop(0, 256, make_scan(hist_a, hist_b), jnp.int32(0))

            def p2(i, c):
                for u in range(U):
                    a0 = i * 16 * U + u * 16
                    pay = pay_b[pl.ds(a0, 16)]
                    q = fetch_add(hist_a, slot_of(pdig(pay)))
                    ka = jnp.bitwise_and(pay, jnp.int32(KA))
                    pay2 = jnp.bitwise_or((pay >> 23) << 15, ka)
                    plsc.store_scatter(pay_a, [remap(q)], pay2)
                return c
            lax.fori_loop(0, V // U, p2, 0)

            lax.fori_loop(0, V // U, make_hsweep(pay_a, hist_b), 0)
            lax.fori_loop(0, 256, make_scan(hist_b, hist_a), jnp.int32(0))

            def p3(i, c):
                for u in range(U):
                    a0 = i * 16 * U + u * 16
                    pay = pay_a[pl.ds(a0, 16)]
                    q = fetch_add(hist_b, slot_of(pdig(pay)))
                    ka = jnp.bitwise_and(pay, jnp.int32(KA))
                    orig = jnp.bitwise_or(
                        (jnp.bitwise_and(ka, jnp.int32(15))) << 11, ka >> 4)
                    plsc.store_scatter(pay_b, [q], orig)
                return c
            lax.fori_loop(0, V // U, p3, 0)

            if rr + 1 < ROWS_PER_W:
                lax.fori_loop(0, 256 // U, zero_hist(hist_b), 0)

            ocopy = pltpu.async_copy(pay_b, out_hbm.at[r], osem)

        ocopy.wait()

    return body(keys_s)


def kernel(x):
    keys = _make_keys(x)
    keys_s = keys.reshape(B, 16, V).swapaxes(1, 2).reshape(B, N)
    return _sc_argsort(keys_s)
